# Initial kernel scaffold; baseline (speedup 1.0000x reference)
import jax
import jax.numpy as jnp
from jax.experimental import pallas as pl


def kernel(input_tensor, embeddings_tensor):
    raise NotImplementedError("write your pallas kernel here")


# speedup vs baseline: 1.4681x; 1.4681x over previous
import functools

import jax
import jax.numpy as jnp
from jax import lax
from jax.experimental import pallas as pl
from jax.experimental.pallas import tpu as pltpu
from jax.experimental.pallas import tpu_sc as plsc

_D = 32
_NW = 32
_CHUNK = 1280


@functools.lru_cache(maxsize=None)
def _build(n_rows: int):
    per_w = n_rows // _NW
    n_chunks = per_w // _CHUNK
    mesh = plsc.VectorSubcoreMesh(core_axis_name="c", subcore_axis_name="s")

    @functools.partial(
        pl.kernel,
        mesh=mesh,
        out_type=jax.ShapeDtypeStruct((n_rows, _D), jnp.float32),
        compiler_params=pltpu.CompilerParams(use_tc_tiling_on_sc=False),
        scratch_types=[
            pltpu.VMEM((_CHUNK,), jnp.int32),
            pltpu.VMEM((_CHUNK, _D), jnp.float32),
            pltpu.SemaphoreType.DMA,
        ],
    )
    def gather_kernel(idx_hbm, table_hbm, out_hbm, idx_v, rows_v, sem):
        wid = lax.axis_index("s") * 2 + lax.axis_index("c")
        base = wid * per_w

        def body(i, carry):
            off = pl.multiple_of(base + i * _CHUNK, 8)
            pltpu.sync_copy(idx_hbm.at[pl.ds(off, _CHUNK)], idx_v)
            pltpu.async_copy(table_hbm.at[idx_v], rows_v, sem).wait()
            pltpu.sync_copy(rows_v, out_hbm.at[pl.ds(off, _CHUNK)])
            return carry

        lax.fori_loop(0, n_chunks, body, 0)

    return gather_kernel


def kernel(input_tensor, embeddings_tensor):
    b, t = input_tensor.shape
    idx = input_tensor.reshape(-1)
    out = _build(b * t)(idx, embeddings_tensor)
    return out.reshape(b, t, _D)


# speedup vs baseline: 1.5011x; 1.0225x over previous
import functools

import jax
import jax.numpy as jnp
from jax import lax
from jax.experimental import pallas as pl
from jax.experimental.pallas import tpu as pltpu
from jax.experimental.pallas import tpu_sc as plsc

_D = 32
_NW = 32
_C = 1024
_NBUF = 3


@functools.lru_cache(maxsize=None)
def _build(n_rows: int):
    per_w = n_rows // _NW
    n_chunks = per_w // _C
    full_rounds = (n_chunks - _NBUF) // _NBUF
    mesh = plsc.VectorSubcoreMesh(core_axis_name="c", subcore_axis_name="s")

    @functools.partial(
        pl.kernel,
        mesh=mesh,
        out_type=jax.ShapeDtypeStruct((n_rows, _D), jnp.float32),
        compiler_params=pltpu.CompilerParams(use_tc_tiling_on_sc=False),
        scratch_types=[
            pltpu.VMEM((per_w,), jnp.int32),
            pltpu.VMEM((_NBUF, _C, _D), jnp.float32),
            pltpu.SemaphoreType.DMA((_NBUF,)),
            pltpu.SemaphoreType.DMA((_NBUF,)),
        ],
    )
    def gather_kernel(idx_hbm, table_hbm, out_hbm, idx_v, rows_v, sem_g, sem_w):
        wid = lax.axis_index("s") * 2 + lax.axis_index("c")
        base = wid * per_w

        def fire_g(i, b):
            pltpu.async_copy(
                table_hbm.at[idx_v.at[pl.ds(i * _C, _C)]],
                rows_v.at[b], sem_g.at[b])

        def wait_g(i, b):
            pltpu.make_async_copy(
                table_hbm.at[idx_v.at[pl.ds(i * _C, _C)]],
                rows_v.at[b], sem_g.at[b]).wait()

        def fire_w(i, b):
            off = pl.multiple_of(base + i * _C, 8)
            pltpu.async_copy(rows_v.at[b], out_hbm.at[pl.ds(off, _C)],
                             sem_w.at[b])

        def wait_w(i, b):
            off = pl.multiple_of(base + i * _C, 8)
            pltpu.make_async_copy(rows_v.at[b], out_hbm.at[pl.ds(off, _C)],
                                  sem_w.at[b]).wait()

        pltpu.sync_copy(idx_hbm.at[pl.ds(pl.multiple_of(base, 8), per_w)],
                        idx_v)

        for b in range(_NBUF):
            fire_g(b, b)

        def body(r, carry):
            for b in range(_NBUF):
                i = r * _NBUF + b
                wait_g(i, b)
                fire_w(i, b)
                wait_w(i, b)
                fire_g(i + _NBUF, b)
            return carry

        lax.fori_loop(0, full_rounds, body, 0)

        for i in range(full_rounds * _NBUF, n_chunks):
            b = i % _NBUF
            wait_g(i, b)
            fire_w(i, b)
            if i + _NBUF < n_chunks:
                wait_w(i, b)
                fire_g(i + _NBUF, b)
        for i in range(max(full_rounds * _NBUF, n_chunks - _NBUF), n_chunks):
            wait_w(i, i % _NBUF)

    return gather_kernel


def kernel(input_tensor, embeddings_tensor):
    b, t = input_tensor.shape
    idx = input_tensor.reshape(-1)
    out = _build(b * t)(idx, embeddings_tensor)
    return out.reshape(b, t, _D)


# speedup vs baseline: 1.5033x; 1.0015x over previous
import functools

import jax
import jax.numpy as jnp
from jax import lax
from jax.experimental import pallas as pl
from jax.experimental.pallas import tpu as pltpu
from jax.experimental.pallas import tpu_sc as plsc

_D = 32
_NW = 32
_C = 512
_NBUF = 6


@functools.lru_cache(maxsize=None)
def _build(n_rows: int):
    per_w = n_rows // _NW
    n_chunks = per_w // _C
    full_rounds = (n_chunks - _NBUF) // _NBUF
    mesh = plsc.VectorSubcoreMesh(core_axis_name="c", subcore_axis_name="s")

    @functools.partial(
        pl.kernel,
        mesh=mesh,
        out_type=jax.ShapeDtypeStruct((n_rows, _D), jnp.float32),
        compiler_params=pltpu.CompilerParams(use_tc_tiling_on_sc=False),
        scratch_types=[
            pltpu.VMEM((per_w,), jnp.int32),
            pltpu.VMEM((_NBUF, _C, _D), jnp.float32),
            pltpu.SemaphoreType.DMA((_NBUF,)),
            pltpu.SemaphoreType.DMA((_NBUF,)),
        ],
    )
    def gather_kernel(idx_hbm, table_hbm, out_hbm, idx_v, rows_v, sem_g, sem_w):
        wid = lax.axis_index("s") * 2 + lax.axis_index("c")
        base = wid * per_w

        def fire_g(i, b):
            pltpu.async_copy(
                table_hbm.at[idx_v.at[pl.ds(i * _C, _C)]],
                rows_v.at[b], sem_g.at[b])

        def wait_g(i, b):
            pltpu.make_async_copy(
                table_hbm.at[idx_v.at[pl.ds(i * _C, _C)]],
                rows_v.at[b], sem_g.at[b]).wait()

        def fire_w(i, b):
            off = pl.multiple_of(base + i * _C, 8)
            pltpu.async_copy(rows_v.at[b], out_hbm.at[pl.ds(off, _C)],
                             sem_w.at[b])

        def wait_w(i, b):
            off = pl.multiple_of(base + i * _C, 8)
            pltpu.make_async_copy(rows_v.at[b], out_hbm.at[pl.ds(off, _C)],
                                  sem_w.at[b]).wait()

        pltpu.sync_copy(idx_hbm.at[pl.ds(pl.multiple_of(base, 8), per_w)],
                        idx_v)

        for b in range(_NBUF):
            fire_g(b, b)

        def body(r, carry):
            for b in range(_NBUF):
                i = r * _NBUF + b
                wait_g(i, b)
                fire_w(i, b)
                wait_w(i, b)
                fire_g(i + _NBUF, b)
            return carry

        lax.fori_loop(0, full_rounds, body, 0)

        for i in range(full_rounds * _NBUF, n_chunks):
            b = i % _NBUF
            wait_g(i, b)
            fire_w(i, b)
            if i + _NBUF < n_chunks:
                wait_w(i, b)
                fire_g(i + _NBUF, b)
        for i in range(max(full_rounds * _NBUF, n_chunks - _NBUF), n_chunks):
            wait_w(i, i % _NBUF)

    return gather_kernel


def kernel(input_tensor, embeddings_tensor):
    b, t = input_tensor.shape
    idx = input_tensor.reshape(-1)
    out = _build(b * t)(idx, embeddings_tensor)
    return out.reshape(b, t, _D)
